# Initial kernel scaffold; baseline (speedup 1.0000x reference)
import jax
import jax.numpy as jnp
from jax.experimental import pallas as pl


def kernel(node_embeddings, neighbour_distances, edge_embedding, edge_index, atomic_numbers, W_pre, bessel_freqs, W1, b1, W2, b2, W3, b3, W_post_0e, W_post_1o, W_self):
    raise NotImplementedError("write your pallas kernel here")


# speedup vs baseline: 1.7211x; 1.7211x over previous
import functools
import math

import jax
import jax.numpy as jnp
from jax import lax
from jax.experimental import pallas as pl
from jax.experimental.pallas import tpu as pltpu
from jax.experimental.pallas import tpu_sc as plsc

N = 10000
E = 320000
CUTOFF = 5.0
F32 = jnp.float32

NTILE = 16
NCORE = 2
EPT = E // NTILE
CHUNK = 128
NFULL = EPT // CHUNK
TAIL = EPT - NFULL * CHUNK
NPAD = 10240
ROWS_PT = NPAD // NTILE
ROW_STEP = 128

BN = 2000
BE = 2000


def _pre_body(ne_ref, w_ref, h_ref):
    hh = jnp.dot(ne_ref[...], w_ref[...], preferred_element_type=F32)
    h_ref[:, 0:64] = hh
    h_ref[:, 64:128] = hh


def _edge_body(r_ref, ee_ref, fp_ref, w1_ref, b1_ref, w2_ref, b2_ref,
               w3_ref, b3_ref, c_ref):
    r = r_ref[...]
    s = jnp.sin(r * fp_ref[...])
    bes = (math.sqrt(2.0 / CUTOFF) * s) / r
    d = r * (1.0 / CUTOFF)
    d2 = d * d
    d6 = d2 * d2 * d2
    env = 1.0 - 28.0 * d6 + 48.0 * d6 * d - 21.0 * d6 * d2
    env = jnp.where(d < 1.0, env, 0.0)
    x = bes * env
    x = jax.nn.silu(jnp.dot(x, w1_ref[...], preferred_element_type=F32)
                    + b1_ref[...])
    x = jax.nn.silu(jnp.dot(x, w2_ref[...], preferred_element_type=F32)
                    + b2_ref[...])
    w = jnp.dot(x, w3_ref[...], preferred_element_type=F32) + b3_ref[...]
    w0 = w[:, :64]
    w1v = w[:, 64:]
    ee = ee_ref[...]
    c_ref[0, :, 0:64] = w0 * ee[:, 0:1]
    c_ref[0, :, 64:128] = w1v * ee[:, 1:2]
    c_ref[1, :, 0:64] = w1v * ee[:, 2:3]
    c_ref[1, :, 64:128] = w1v * ee[:, 3:4]


def _sc_body(h_hbm, c_hbm, cen_hbm, nei_hbm, t_hbm,
             acc, cidx, nidx, crows, msg,
             cidx_t, nidx_t, crows_t, msg_t, sem):
    c = lax.axis_index("c")
    s = lax.axis_index("s")

    def zrow(i, carry):
        for q in range(8):
            msg[i, pl.ds(q * 16, 16)] = jnp.zeros((16,), F32)
        return carry
    lax.fori_loop(0, CHUNK, zrow, 0)
    rb = s * ROWS_PT
    for j in range(ROWS_PT // ROW_STEP):
        pltpu.sync_copy(msg.at[pl.ds(0, ROW_STEP)],
                        acc.at[pl.ds(rb + j * ROW_STEP, ROW_STEP)])
    plsc.subcore_barrier()

    base0 = s * EPT

    def do_chunk(base, n, ci, ni, cr, mg):
        pltpu.sync_copy(cen_hbm.at[pl.ds(base, n)], ci)
        pltpu.sync_copy(nei_hbm.at[pl.ds(base, n)], ni)
        pltpu.async_copy(h_hbm.at[ni], mg, sem).wait()
        pltpu.sync_copy(c_hbm.at[c, pl.ds(base, n)], cr)

        def body(e, carry):
            for q in range(8):
                mg[e, pl.ds(16 * q, 16)] = (cr[e, pl.ds(16 * q, 16)]
                                            * mg[e, pl.ds(16 * q, 16)])
            return carry
        lax.fori_loop(0, n, body, 0)
        pltpu.sync_copy(mg, acc.at[ci], add=True)

    def chunk_loop(k, carry):
        do_chunk(base0 + k * CHUNK, CHUNK, cidx, nidx, crows, msg)
        return carry
    lax.fori_loop(0, NFULL, chunk_loop, 0)
    if TAIL:
        do_chunk(base0 + NFULL * CHUNK, TAIL, cidx_t, nidx_t, crows_t, msg_t)

    plsc.subcore_barrier()
    for j in range(ROWS_PT // ROW_STEP):
        pltpu.sync_copy(acc.at[pl.ds(rb + j * ROW_STEP, ROW_STEP)],
                        msg.at[pl.ds(0, ROW_STEP)])
        pltpu.sync_copy(msg.at[pl.ds(0, ROW_STEP)],
                        t_hbm.at[c, pl.ds(rb + j * ROW_STEP, ROW_STEP)])


def _sc_call(h, coef, central, neigh):
    mesh = plsc.VectorSubcoreMesh(core_axis_name="c", subcore_axis_name="s")
    return pl.kernel(
        _sc_body,
        out_type=jax.ShapeDtypeStruct((NCORE, NPAD, 128), F32),
        mesh=mesh,
        scratch_types=[
            pltpu.VMEM_SHARED((NPAD, 128), F32),
            pltpu.VMEM((CHUNK,), jnp.int32),
            pltpu.VMEM((CHUNK,), jnp.int32),
            pltpu.VMEM((CHUNK, 128), F32),
            pltpu.VMEM((CHUNK, 128), F32),
            pltpu.VMEM((TAIL,), jnp.int32),
            pltpu.VMEM((TAIL,), jnp.int32),
            pltpu.VMEM((TAIL, 128), F32),
            pltpu.VMEM((TAIL, 128), F32),
            pltpu.SemaphoreType.DMA,
        ],
    )(h, coef, central, neigh)


def _post_body(ta_ref, tb_ref, ne_ref, z_ref, w0e_ref, w1o_ref, wself_ref,
               s_ref, gx_ref, gy_ref, gz_ref):
    ta = ta_ref[0, :, :]
    tb = tb_ref[0, :, :]
    t0e = ta[:, :64]
    t1x = ta[:, 64:]
    t1y = tb[:, :64]
    t1z = tb[:, 64:]
    post0 = jnp.dot(t0e, w0e_ref[...], preferred_element_type=F32)
    si_all = jnp.dot(ne_ref[...], wself_ref[...], preferred_element_type=F32)
    z = z_ref[...]
    si = jnp.where(z == 0, si_all[:, 0:128], 0.0)
    si = si + jnp.where(z == 1, si_all[:, 128:256], 0.0)
    si = si + jnp.where(z == 2, si_all[:, 256:384], 0.0)
    si = si + jnp.where(z == 3, si_all[:, 384:512], 0.0)
    new0 = post0 + si
    sc = jax.nn.silu(new0[:, :64])
    g = jax.nn.silu(new0[:, 64:])
    w1o = w1o_ref[...]
    s_ref[...] = sc
    gx_ref[...] = g * jnp.dot(t1x, w1o, preferred_element_type=F32)
    gy_ref[...] = g * jnp.dot(t1y, w1o, preferred_element_type=F32)
    gz_ref[...] = g * jnp.dot(t1z, w1o, preferred_element_type=F32)


def kernel(node_embeddings, neighbour_distances, edge_embedding, edge_index,
           atomic_numbers, W_pre, bessel_freqs, W1, b1, W2, b2, W3, b3,
           W_post_0e, W_post_1o, W_self):
    central = edge_index[0]
    neigh = edge_index[1]

    h = pl.pallas_call(
        _pre_body,
        grid=(N // BN,),
        in_specs=[
            pl.BlockSpec((BN, 64), lambda i: (i, 0)),
            pl.BlockSpec((64, 64), lambda i: (0, 0)),
        ],
        out_specs=pl.BlockSpec((BN, 128), lambda i: (i, 0)),
        out_shape=jax.ShapeDtypeStruct((N, 128), F32),
    )(node_embeddings, W_pre * 0.125)

    w1p = jnp.zeros((128, 128), F32).at[:8, :10].set(W1)
    b1p = jnp.zeros((1, 128), F32).at[0, :10].set(b1)
    w2p = jnp.zeros((128, 128), F32).at[:10, :10].set(W2)
    b2p = jnp.zeros((1, 128), F32).at[0, :10].set(b2)
    w3p = jnp.zeros((128, 128), F32).at[:10, :].set(W3)
    b3p = b3.reshape(1, 128)
    freqs_p = jnp.zeros((1, 128), F32).at[0, :8].set(bessel_freqs / CUTOFF)
    r2 = neighbour_distances.reshape(E, 1)

    coef = pl.pallas_call(
        _edge_body,
        grid=(E // BE,),
        in_specs=[
            pl.BlockSpec((BE, 1), lambda i: (i, 0)),
            pl.BlockSpec((BE, 4), lambda i: (i, 0)),
            pl.BlockSpec((1, 128), lambda i: (0, 0)),
            pl.BlockSpec((128, 128), lambda i: (0, 0)),
            pl.BlockSpec((1, 128), lambda i: (0, 0)),
            pl.BlockSpec((128, 128), lambda i: (0, 0)),
            pl.BlockSpec((1, 128), lambda i: (0, 0)),
            pl.BlockSpec((128, 128), lambda i: (0, 0)),
            pl.BlockSpec((1, 128), lambda i: (0, 0)),
        ],
        out_specs=pl.BlockSpec((NCORE, BE, 128), lambda i: (0, i, 0)),
        out_shape=jax.ShapeDtypeStruct((NCORE, E, 128), F32),
    )(r2, edge_embedding, freqs_p, w1p, b1p, w2p, b2p, w3p, b3p)

    tot = _sc_call(h, coef, central, neigh)[:, :N, :]

    w0e_s = W_post_0e * 0.125
    w1o_s = W_post_1o * 0.125
    wself_s = W_self.reshape(64, 512) * (1.0 / 16.0)
    z2 = atomic_numbers.reshape(N, 1)

    outs = pl.pallas_call(
        _post_body,
        grid=(N // BN,),
        in_specs=[
            pl.BlockSpec((1, BN, 128), lambda i: (0, i, 0)),
            pl.BlockSpec((1, BN, 128), lambda i: (1, i, 0)),
            pl.BlockSpec((BN, 64), lambda i: (i, 0)),
            pl.BlockSpec((BN, 1), lambda i: (i, 0)),
            pl.BlockSpec((64, 128), lambda i: (0, 0)),
            pl.BlockSpec((64, 64), lambda i: (0, 0)),
            pl.BlockSpec((64, 512), lambda i: (0, 0)),
        ],
        out_specs=[
            pl.BlockSpec((BN, 64), lambda i: (i, 0)),
            pl.BlockSpec((BN, 64), lambda i: (i, 0)),
            pl.BlockSpec((BN, 64), lambda i: (i, 0)),
            pl.BlockSpec((BN, 64), lambda i: (i, 0)),
        ],
        out_shape=[jax.ShapeDtypeStruct((N, 64), F32)] * 4,
    )(tot, tot, node_embeddings, z2, w0e_s, w1o_s, wself_s)
    sc, gx, gy, gz = outs

    gated = jnp.stack([gx, gy, gz], axis=-1).reshape(N, 192)
    return jnp.concatenate([sc, gated], axis=1)


# speedup vs baseline: 2.2268x; 1.2938x over previous
import functools
import math

import jax
import jax.numpy as jnp
from jax import lax
from jax.experimental import pallas as pl
from jax.experimental.pallas import tpu as pltpu
from jax.experimental.pallas import tpu_sc as plsc

N = 10000
E = 320000
CUTOFF = 5.0
F32 = jnp.float32

NTILE = 16
NCORE = 2
NPART = 4
EP = E // NPART
EPT = EP // NTILE
CHUNK = 128
NFULL = EPT // CHUNK
TAIL = EPT - NFULL * CHUNK
NPAD = 10240
ROWS_PT = NPAD // NTILE
ROW_STEP = 128

BN = 2000
BE = 2000


def _pre_body(ne_ref, w_ref, h_ref):
    hh = jnp.dot(ne_ref[...], w_ref[...], preferred_element_type=F32)
    h_ref[:, 0:64] = hh
    h_ref[:, 64:128] = hh


def _edge_body(r_ref, ee_ref, fp_ref, w1_ref, b1_ref, w2_ref, b2_ref,
               w3_ref, b3_ref, c_ref):
    r = r_ref[...]
    s = jnp.sin(r * fp_ref[...])
    bes = (math.sqrt(2.0 / CUTOFF) * s) / r
    d = r * (1.0 / CUTOFF)
    d2 = d * d
    d6 = d2 * d2 * d2
    env = 1.0 - 28.0 * d6 + 48.0 * d6 * d - 21.0 * d6 * d2
    env = jnp.where(d < 1.0, env, 0.0)
    x = bes * env
    x = jax.nn.silu(jnp.dot(x, w1_ref[...], preferred_element_type=F32)
                    + b1_ref[...])
    x = jax.nn.silu(jnp.dot(x, w2_ref[...], preferred_element_type=F32)
                    + b2_ref[...])
    w = jnp.dot(x, w3_ref[...], preferred_element_type=F32) + b3_ref[...]
    w0 = w[:, :64]
    w1v = w[:, 64:]
    ee = ee_ref[...]
    c_ref[0, :, 0:64] = w0 * ee[:, 0:1]
    c_ref[0, :, 64:128] = w1v * ee[:, 1:2]
    c_ref[1, :, 0:64] = w1v * ee[:, 2:3]
    c_ref[1, :, 64:128] = w1v * ee[:, 3:4]


def _sc_body(h_hbm, c_hbm, cen_hbm, nei_hbm, t_hbm,
             acc, cidx, nidx, crows, msg,
             cidx_t, nidx_t, crows_t, msg_t, sem):
    c = lax.axis_index("c")
    s = lax.axis_index("s")

    def zrow(i, carry):
        for q in range(8):
            msg[i, pl.ds(q * 16, 16)] = jnp.zeros((16,), F32)
        return carry
    lax.fori_loop(0, CHUNK, zrow, 0)
    rb = s * ROWS_PT
    for j in range(ROWS_PT // ROW_STEP):
        pltpu.sync_copy(msg.at[pl.ds(0, ROW_STEP)],
                        acc.at[pl.ds(rb + j * ROW_STEP, ROW_STEP)])
    plsc.subcore_barrier()

    base0 = s * EPT

    def do_chunk(base, n, ci, ni, cr, mg):
        pltpu.sync_copy(cen_hbm.at[pl.ds(base, n)], ci)
        pltpu.sync_copy(nei_hbm.at[pl.ds(base, n)], ni)
        pltpu.async_copy(h_hbm.at[ni], mg, sem).wait()
        pltpu.sync_copy(c_hbm.at[c, pl.ds(base, n)], cr)

        def body(e, carry):
            for q in range(8):
                mg[e, pl.ds(16 * q, 16)] = (cr[e, pl.ds(16 * q, 16)]
                                            * mg[e, pl.ds(16 * q, 16)])
            return carry
        lax.fori_loop(0, n, body, 0)
        pltpu.sync_copy(mg, acc.at[ci], add=True)

    def chunk_loop(k, carry):
        do_chunk(base0 + k * CHUNK, CHUNK, cidx, nidx, crows, msg)
        return carry
    lax.fori_loop(0, NFULL, chunk_loop, 0)
    if TAIL:
        do_chunk(base0 + NFULL * CHUNK, TAIL, cidx_t, nidx_t, crows_t, msg_t)

    plsc.subcore_barrier()
    for j in range(ROWS_PT // ROW_STEP):
        pltpu.sync_copy(acc.at[pl.ds(rb + j * ROW_STEP, ROW_STEP)],
                        msg.at[pl.ds(0, ROW_STEP)])
        pltpu.sync_copy(msg.at[pl.ds(0, ROW_STEP)],
                        t_hbm.at[c, pl.ds(rb + j * ROW_STEP, ROW_STEP)])


def _sc_call(h, coef, central, neigh):
    mesh = plsc.VectorSubcoreMesh(core_axis_name="c", subcore_axis_name="s")
    return pl.kernel(
        _sc_body,
        out_type=jax.ShapeDtypeStruct((NCORE, NPAD, 128), F32),
        mesh=mesh,
        scratch_types=[
            pltpu.VMEM_SHARED((NPAD, 128), F32),
            pltpu.VMEM((CHUNK,), jnp.int32),
            pltpu.VMEM((CHUNK,), jnp.int32),
            pltpu.VMEM((CHUNK, 128), F32),
            pltpu.VMEM((CHUNK, 128), F32),
            pltpu.VMEM((TAIL,), jnp.int32),
            pltpu.VMEM((TAIL,), jnp.int32),
            pltpu.VMEM((TAIL, 128), F32),
            pltpu.VMEM((TAIL, 128), F32),
            pltpu.SemaphoreType.DMA,
        ],
    )(h, coef, central, neigh)


def _post_body(p0_ref, p1_ref, p2_ref, p3_ref, ne_ref, z_ref, w0e_ref,
               w1o_ref, wself_ref, s_ref, gx_ref, gy_ref, gz_ref):
    ta = (p0_ref[0, :, :] + p1_ref[0, :, :]
          + p2_ref[0, :, :] + p3_ref[0, :, :])
    tb = (p0_ref[1, :, :] + p1_ref[1, :, :]
          + p2_ref[1, :, :] + p3_ref[1, :, :])
    t0e = ta[:, :64]
    t1x = ta[:, 64:]
    t1y = tb[:, :64]
    t1z = tb[:, 64:]
    post0 = jnp.dot(t0e, w0e_ref[...], preferred_element_type=F32)
    si_all = jnp.dot(ne_ref[...], wself_ref[...], preferred_element_type=F32)
    z = z_ref[...]
    si = jnp.where(z == 0, si_all[:, 0:128], 0.0)
    si = si + jnp.where(z == 1, si_all[:, 128:256], 0.0)
    si = si + jnp.where(z == 2, si_all[:, 256:384], 0.0)
    si = si + jnp.where(z == 3, si_all[:, 384:512], 0.0)
    new0 = post0 + si
    sc = jax.nn.silu(new0[:, :64])
    g = jax.nn.silu(new0[:, 64:])
    w1o = w1o_ref[...]
    s_ref[...] = sc
    gx_ref[...] = g * jnp.dot(t1x, w1o, preferred_element_type=F32)
    gy_ref[...] = g * jnp.dot(t1y, w1o, preferred_element_type=F32)
    gz_ref[...] = g * jnp.dot(t1z, w1o, preferred_element_type=F32)


def kernel(node_embeddings, neighbour_distances, edge_embedding, edge_index,
           atomic_numbers, W_pre, bessel_freqs, W1, b1, W2, b2, W3, b3,
           W_post_0e, W_post_1o, W_self):
    central = edge_index[0]
    neigh = edge_index[1]

    h = pl.pallas_call(
        _pre_body,
        grid=(N // BN,),
        in_specs=[
            pl.BlockSpec((BN, 64), lambda i: (i, 0)),
            pl.BlockSpec((64, 64), lambda i: (0, 0)),
        ],
        out_specs=pl.BlockSpec((BN, 128), lambda i: (i, 0)),
        out_shape=jax.ShapeDtypeStruct((N, 128), F32),
    )(node_embeddings, W_pre * 0.125)

    w1p = jnp.zeros((128, 128), F32).at[:8, :10].set(W1)
    b1p = jnp.zeros((1, 128), F32).at[0, :10].set(b1)
    w2p = jnp.zeros((128, 128), F32).at[:10, :10].set(W2)
    b2p = jnp.zeros((1, 128), F32).at[0, :10].set(b2)
    w3p = jnp.zeros((128, 128), F32).at[:10, :].set(W3)
    b3p = b3.reshape(1, 128)
    freqs_p = jnp.zeros((1, 128), F32).at[0, :8].set(bessel_freqs / CUTOFF)
    r2 = neighbour_distances.reshape(E, 1)

    def coef_part(i):
        return pl.pallas_call(
            _edge_body,
            grid=(EP // BE,),
            in_specs=[
                pl.BlockSpec((BE, 1), lambda i: (i, 0)),
                pl.BlockSpec((BE, 4), lambda i: (i, 0)),
                pl.BlockSpec((1, 128), lambda i: (0, 0)),
                pl.BlockSpec((128, 128), lambda i: (0, 0)),
                pl.BlockSpec((1, 128), lambda i: (0, 0)),
                pl.BlockSpec((128, 128), lambda i: (0, 0)),
                pl.BlockSpec((1, 128), lambda i: (0, 0)),
                pl.BlockSpec((128, 128), lambda i: (0, 0)),
                pl.BlockSpec((1, 128), lambda i: (0, 0)),
            ],
            out_specs=pl.BlockSpec((NCORE, BE, 128), lambda i: (0, i, 0)),
            out_shape=jax.ShapeDtypeStruct((NCORE, EP, 128), F32),
        )(r2[i * EP:(i + 1) * EP], edge_embedding[i * EP:(i + 1) * EP],
          freqs_p, w1p, b1p, w2p, b2p, w3p, b3p)

    parts = []
    for i in range(NPART):
        coef_i = coef_part(i)
        parts.append(_sc_call(h, coef_i,
                              central[i * EP:(i + 1) * EP],
                              neigh[i * EP:(i + 1) * EP]))

    w0e_s = W_post_0e * 0.125
    w1o_s = W_post_1o * 0.125
    wself_s = W_self.reshape(64, 512) * (1.0 / 16.0)
    z2 = atomic_numbers.reshape(N, 1)

    part_spec = pl.BlockSpec((NCORE, BN, 128), lambda i: (0, i, 0))
    outs = pl.pallas_call(
        _post_body,
        grid=(N // BN,),
        in_specs=[
            part_spec,
            part_spec,
            part_spec,
            part_spec,
            pl.BlockSpec((BN, 64), lambda i: (i, 0)),
            pl.BlockSpec((BN, 1), lambda i: (i, 0)),
            pl.BlockSpec((64, 128), lambda i: (0, 0)),
            pl.BlockSpec((64, 64), lambda i: (0, 0)),
            pl.BlockSpec((64, 512), lambda i: (0, 0)),
        ],
        out_specs=[
            pl.BlockSpec((BN, 64), lambda i: (i, 0)),
            pl.BlockSpec((BN, 64), lambda i: (i, 0)),
            pl.BlockSpec((BN, 64), lambda i: (i, 0)),
            pl.BlockSpec((BN, 64), lambda i: (i, 0)),
        ],
        out_shape=[jax.ShapeDtypeStruct((N, 64), F32)] * 4,
    )(parts[0], parts[1], parts[2], parts[3],
      node_embeddings, z2, w0e_s, w1o_s, wself_s)
    sc, gx, gy, gz = outs

    gated = jnp.stack([gx, gy, gz], axis=-1).reshape(N, 192)
    return jnp.concatenate([sc, gated], axis=1)


# speedup vs baseline: 2.6469x; 1.1887x over previous
import functools
import math

import jax
import jax.numpy as jnp
from jax import lax
from jax.experimental import pallas as pl
from jax.experimental.pallas import tpu as pltpu
from jax.experimental.pallas import tpu_sc as plsc

N = 10000
E = 320000
CUTOFF = 5.0
F32 = jnp.float32

NTILE = 16
NCORE = 2
NPART = 4
EP = E // NPART
EPT = EP // NTILE
CHUNK = 128
NFULL = EPT // CHUNK
TAIL = EPT - NFULL * CHUNK
NPAD = 10240
ROWS_PT = NPAD // NTILE
ROW_STEP = 128

BN = 2000
BE = 2000


def _pre_body(ne_ref, w_ref, h_ref):
    hh = jnp.dot(ne_ref[...], w_ref[...], preferred_element_type=F32)
    h_ref[:, 0:64] = hh
    h_ref[:, 64:128] = hh


def _edge_body(pk_ref, fp_ref, w1_ref, b1_ref, w2_ref, b2_ref,
               w3_ref, b3_ref, c_ref):
    t8 = jnp.transpose(pk_ref[...])
    r = t8[:, 0:1]
    s = jnp.sin(r * fp_ref[...])
    bes = (math.sqrt(2.0 / CUTOFF) * s) / r
    d = r * (1.0 / CUTOFF)
    d2 = d * d
    d6 = d2 * d2 * d2
    env = 1.0 - 28.0 * d6 + 48.0 * d6 * d - 21.0 * d6 * d2
    env = jnp.where(d < 1.0, env, 0.0)
    x = bes * env
    x = jax.nn.silu(jnp.dot(x, w1_ref[...], preferred_element_type=F32)
                    + b1_ref[...])
    x = jax.nn.silu(jnp.dot(x, w2_ref[...], preferred_element_type=F32)
                    + b2_ref[...])
    w = jnp.dot(x, w3_ref[...], preferred_element_type=F32) + b3_ref[...]
    w0 = w[:, :64]
    w1v = w[:, 64:]
    ee = t8[:, 1:5]
    c_ref[0, :, 0:64] = w0 * ee[:, 0:1]
    c_ref[0, :, 64:128] = w1v * ee[:, 1:2]
    c_ref[1, :, 0:64] = w1v * ee[:, 2:3]
    c_ref[1, :, 64:128] = w1v * ee[:, 3:4]


def _sc_body(pbase, h_hbm, c_hbm, cen_hbm, nei_hbm, t_hbm,
             acc, cidx, nidx, crows, msg,
             cidx_t, nidx_t, crows_t, msg_t, sem):
    c = lax.axis_index("c")
    s = lax.axis_index("s")

    def zrow(i, carry):
        for q in range(8):
            msg[i, pl.ds(q * 16, 16)] = jnp.zeros((16,), F32)
        return carry
    lax.fori_loop(0, CHUNK, zrow, 0)
    rb = s * ROWS_PT
    for j in range(ROWS_PT // ROW_STEP):
        pltpu.sync_copy(msg.at[pl.ds(0, ROW_STEP)],
                        acc.at[pl.ds(rb + j * ROW_STEP, ROW_STEP)])
    plsc.subcore_barrier()

    base0 = s * EPT

    def do_chunk(base, n, ci, ni, cr, mg):
        pltpu.sync_copy(cen_hbm.at[pl.ds(pbase + base, n)], ci)
        pltpu.sync_copy(nei_hbm.at[pl.ds(pbase + base, n)], ni)
        pltpu.async_copy(h_hbm.at[ni], mg, sem).wait()
        pltpu.sync_copy(c_hbm.at[c, pl.ds(base, n)], cr)

        def body(e, carry):
            for q in range(8):
                mg[e, pl.ds(16 * q, 16)] = (cr[e, pl.ds(16 * q, 16)]
                                            * mg[e, pl.ds(16 * q, 16)])
            return carry
        lax.fori_loop(0, n, body, 0)
        pltpu.sync_copy(mg, acc.at[ci], add=True)

    def chunk_loop(k, carry):
        do_chunk(base0 + k * CHUNK, CHUNK, cidx, nidx, crows, msg)
        return carry
    lax.fori_loop(0, NFULL, chunk_loop, 0)
    if TAIL:
        do_chunk(base0 + NFULL * CHUNK, TAIL, cidx_t, nidx_t, crows_t, msg_t)

    plsc.subcore_barrier()
    for j in range(ROWS_PT // ROW_STEP):
        pltpu.sync_copy(acc.at[pl.ds(rb + j * ROW_STEP, ROW_STEP)],
                        msg.at[pl.ds(0, ROW_STEP)])
        pltpu.sync_copy(msg.at[pl.ds(0, ROW_STEP)],
                        t_hbm.at[c, pl.ds(rb + j * ROW_STEP, ROW_STEP)])


def _sc_call(h, coef, central, neigh, pbase):
    mesh = plsc.VectorSubcoreMesh(core_axis_name="c", subcore_axis_name="s")
    return pl.kernel(
        functools.partial(_sc_body, pbase),
        out_type=jax.ShapeDtypeStruct((NCORE, NPAD, 128), F32),
        mesh=mesh,
        scratch_types=[
            pltpu.VMEM_SHARED((NPAD, 128), F32),
            pltpu.VMEM((CHUNK,), jnp.int32),
            pltpu.VMEM((CHUNK,), jnp.int32),
            pltpu.VMEM((CHUNK, 128), F32),
            pltpu.VMEM((CHUNK, 128), F32),
            pltpu.VMEM((TAIL,), jnp.int32),
            pltpu.VMEM((TAIL,), jnp.int32),
            pltpu.VMEM((TAIL, 128), F32),
            pltpu.VMEM((TAIL, 128), F32),
            pltpu.SemaphoreType.DMA,
        ],
    )(h, coef, central, neigh)


def _post_body(p0_ref, p1_ref, p2_ref, p3_ref, ne_ref, z_ref, w0e_ref,
               w1o_ref, wself_ref, s_ref, gx_ref, gy_ref, gz_ref):
    ta = (p0_ref[0, :, :] + p1_ref[0, :, :]
          + p2_ref[0, :, :] + p3_ref[0, :, :])
    tb = (p0_ref[1, :, :] + p1_ref[1, :, :]
          + p2_ref[1, :, :] + p3_ref[1, :, :])
    t0e = ta[:, :64]
    t1x = ta[:, 64:]
    t1y = tb[:, :64]
    t1z = tb[:, 64:]
    post0 = jnp.dot(t0e, w0e_ref[...], preferred_element_type=F32)
    si_all = jnp.dot(ne_ref[...], wself_ref[...], preferred_element_type=F32)
    z = z_ref[...]
    si = jnp.where(z == 0, si_all[:, 0:128], 0.0)
    si = si + jnp.where(z == 1, si_all[:, 128:256], 0.0)
    si = si + jnp.where(z == 2, si_all[:, 256:384], 0.0)
    si = si + jnp.where(z == 3, si_all[:, 384:512], 0.0)
    new0 = post0 + si
    sc = jax.nn.silu(new0[:, :64])
    g = jax.nn.silu(new0[:, 64:])
    w1o = w1o_ref[...]
    s_ref[...] = sc
    gx_ref[...] = g * jnp.dot(t1x, w1o, preferred_element_type=F32)
    gy_ref[...] = g * jnp.dot(t1y, w1o, preferred_element_type=F32)
    gz_ref[...] = g * jnp.dot(t1z, w1o, preferred_element_type=F32)


def kernel(node_embeddings, neighbour_distances, edge_embedding, edge_index,
           atomic_numbers, W_pre, bessel_freqs, W1, b1, W2, b2, W3, b3,
           W_post_0e, W_post_1o, W_self):
    central = edge_index[0]
    neigh = edge_index[1]

    h = pl.pallas_call(
        _pre_body,
        grid=(N // BN,),
        in_specs=[
            pl.BlockSpec((BN, 64), lambda i: (i, 0)),
            pl.BlockSpec((64, 64), lambda i: (0, 0)),
        ],
        out_specs=pl.BlockSpec((BN, 128), lambda i: (i, 0)),
        out_shape=jax.ShapeDtypeStruct((N, 128), F32),
    )(node_embeddings, W_pre * 0.125)

    w1p = jnp.zeros((128, 128), F32).at[:8, :10].set(W1)
    b1p = jnp.zeros((1, 128), F32).at[0, :10].set(b1)
    w2p = jnp.zeros((128, 128), F32).at[:10, :10].set(W2)
    b2p = jnp.zeros((1, 128), F32).at[0, :10].set(b2)
    w3p = jnp.zeros((128, 128), F32).at[:10, :].set(W3)
    b3p = b3.reshape(1, 128)
    freqs_p = jnp.zeros((1, 128), F32).at[0, :8].set(bessel_freqs / CUTOFF)
    nblk = E // BE
    r3 = neighbour_distances.reshape(nblk, 1, BE)
    e3 = edge_embedding.T.reshape(4, nblk, BE).transpose(1, 0, 2)
    packed = jnp.concatenate(
        [r3, e3, jnp.zeros((nblk, 3, BE), F32)], axis=1).reshape(8 * nblk, BE)
    nb = EP // BE

    def coef_part(i):
        return pl.pallas_call(
            _edge_body,
            grid=(nb,),
            in_specs=[
                pl.BlockSpec((8, BE), lambda j, i=i: (i * nb + j, 0)),
                pl.BlockSpec((1, 128), lambda j: (0, 0)),
                pl.BlockSpec((128, 128), lambda j: (0, 0)),
                pl.BlockSpec((1, 128), lambda j: (0, 0)),
                pl.BlockSpec((128, 128), lambda j: (0, 0)),
                pl.BlockSpec((1, 128), lambda j: (0, 0)),
                pl.BlockSpec((128, 128), lambda j: (0, 0)),
                pl.BlockSpec((1, 128), lambda j: (0, 0)),
            ],
            out_specs=pl.BlockSpec((NCORE, BE, 128), lambda j: (0, j, 0)),
            out_shape=jax.ShapeDtypeStruct((NCORE, EP, 128), F32),
        )(packed, freqs_p, w1p, b1p, w2p, b2p, w3p, b3p)

    parts = []
    for i in range(NPART):
        coef_i = coef_part(i)
        parts.append(_sc_call(h, coef_i, central, neigh, i * EP))

    w0e_s = W_post_0e * 0.125
    w1o_s = W_post_1o * 0.125
    wself_s = W_self.reshape(64, 512) * (1.0 / 16.0)
    z2 = atomic_numbers.reshape(N, 1)

    part_spec = pl.BlockSpec((NCORE, BN, 128), lambda i: (0, i, 0))
    outs = pl.pallas_call(
        _post_body,
        grid=(N // BN,),
        in_specs=[
            part_spec,
            part_spec,
            part_spec,
            part_spec,
            pl.BlockSpec((BN, 64), lambda i: (i, 0)),
            pl.BlockSpec((BN, 1), lambda i: (i, 0)),
            pl.BlockSpec((64, 128), lambda i: (0, 0)),
            pl.BlockSpec((64, 64), lambda i: (0, 0)),
            pl.BlockSpec((64, 512), lambda i: (0, 0)),
        ],
        out_specs=[
            pl.BlockSpec((BN, 64), lambda i: (i, 0)),
            pl.BlockSpec((BN, 64), lambda i: (i, 0)),
            pl.BlockSpec((BN, 64), lambda i: (i, 0)),
            pl.BlockSpec((BN, 64), lambda i: (i, 0)),
        ],
        out_shape=[jax.ShapeDtypeStruct((N, 64), F32)] * 4,
    )(parts[0], parts[1], parts[2], parts[3],
      node_embeddings, z2, w0e_s, w1o_s, wself_s)
    sc, gx, gy, gz = outs

    gated = jnp.stack([gx, gy, gz], axis=-1).reshape(N, 192)
    return jnp.concatenate([sc, gated], axis=1)


# speedup vs baseline: 2.7037x; 1.0215x over previous
import functools
import math

import jax
import jax.numpy as jnp
from jax import lax
from jax.experimental import pallas as pl
from jax.experimental.pallas import tpu as pltpu
from jax.experimental.pallas import tpu_sc as plsc

N = 10000
E = 320000
CUTOFF = 5.0
F32 = jnp.float32

NTILE = 16
NCORE = 2
NPART = 4
EP = E // NPART
EPT = EP // NTILE
CHUNK = 128
NFULL = EPT // CHUNK
TAIL = EPT - NFULL * CHUNK
NPAD = 10240
ROWS_PT = NPAD // NTILE
ROW_STEP = 128

BN = 2000
BE = 2000


def _pre_body(ne_ref, w_ref, h_ref):
    hh = jnp.dot(ne_ref[...], w_ref[...], preferred_element_type=F32)
    h_ref[:, 0:64] = hh
    h_ref[:, 64:128] = hh


def _edge_body(pk_ref, fp_ref, w1_ref, b1_ref, w2_ref, b2_ref,
               w3_ref, b3_ref, c_ref):
    t8 = jnp.transpose(pk_ref[...])
    r = t8[:, 0:1]
    s = jnp.sin(r * fp_ref[...])
    bes = (math.sqrt(2.0 / CUTOFF) * s) / r
    d = r * (1.0 / CUTOFF)
    d2 = d * d
    d6 = d2 * d2 * d2
    env = 1.0 - 28.0 * d6 + 48.0 * d6 * d - 21.0 * d6 * d2
    env = jnp.where(d < 1.0, env, 0.0)
    x = bes * env
    x = jax.nn.silu(jnp.dot(x, w1_ref[...], preferred_element_type=F32)
                    + b1_ref[...])
    x = jax.nn.silu(jnp.dot(x, w2_ref[...], preferred_element_type=F32)
                    + b2_ref[...])
    w = jnp.dot(x, w3_ref[...], preferred_element_type=F32) + b3_ref[...]
    w0 = w[:, :64]
    w1v = w[:, 64:]
    ee = t8[:, 1:5]
    c_ref[0, :, 0:64] = w0 * ee[:, 0:1]
    c_ref[0, :, 64:128] = w1v * ee[:, 1:2]
    c_ref[1, :, 0:64] = w1v * ee[:, 2:3]
    c_ref[1, :, 64:128] = w1v * ee[:, 3:4]


def _sc_body(pbase, h_hbm, c_hbm, cen_hbm, nei_hbm, t_hbm,
             acc, cidx, nidx, crows, msg,
             cidx_t, nidx_t, crows_t, msg_t, sem):
    c = lax.axis_index("c")
    s = lax.axis_index("s")

    def zrow(i, carry):
        for q in range(8):
            msg[i, pl.ds(q * 16, 16)] = jnp.zeros((16,), F32)
        return carry
    lax.fori_loop(0, CHUNK, zrow, 0)
    rb = s * ROWS_PT
    for j in range(ROWS_PT // ROW_STEP):
        pltpu.sync_copy(msg.at[pl.ds(0, ROW_STEP)],
                        acc.at[pl.ds(rb + j * ROW_STEP, ROW_STEP)])
    plsc.subcore_barrier()

    base0 = s * EPT

    def do_chunk(base, n, ci, ni, cr, mg):
        pltpu.sync_copy(cen_hbm.at[pl.ds(pbase + base, n)], ci)
        pltpu.sync_copy(nei_hbm.at[pl.ds(pbase + base, n)], ni)
        pltpu.async_copy(h_hbm.at[ni], mg, sem).wait()
        pltpu.sync_copy(c_hbm.at[c, pl.ds(base, n)], cr)

        def body(i2, carry):
            for d in range(2):
                e = i2 * 2 + d
                for q in range(4):
                    hq = mg[e, pl.ds(16 * q, 16)]
                    mg[e, pl.ds(16 * q, 16)] = cr[e, pl.ds(16 * q, 16)] * hq
                    mg[e, pl.ds(16 * q + 64, 16)] = (
                        cr[e, pl.ds(16 * q + 64, 16)] * hq)
            return carry
        lax.fori_loop(0, n // 2, body, 0)
        pltpu.sync_copy(mg, acc.at[ci], add=True)

    def chunk_loop(k, carry):
        do_chunk(base0 + k * CHUNK, CHUNK, cidx, nidx, crows, msg)
        return carry
    lax.fori_loop(0, NFULL, chunk_loop, 0)
    if TAIL:
        do_chunk(base0 + NFULL * CHUNK, TAIL, cidx_t, nidx_t, crows_t, msg_t)

    plsc.subcore_barrier()
    for j in range(ROWS_PT // ROW_STEP):
        pltpu.sync_copy(acc.at[pl.ds(rb + j * ROW_STEP, ROW_STEP)],
                        msg.at[pl.ds(0, ROW_STEP)])
        pltpu.sync_copy(msg.at[pl.ds(0, ROW_STEP)],
                        t_hbm.at[c, pl.ds(rb + j * ROW_STEP, ROW_STEP)])


def _sc_call(h, coef, central, neigh, pbase):
    mesh = plsc.VectorSubcoreMesh(core_axis_name="c", subcore_axis_name="s")
    return pl.kernel(
        functools.partial(_sc_body, pbase),
        out_type=jax.ShapeDtypeStruct((NCORE, NPAD, 128), F32),
        mesh=mesh,
        scratch_types=[
            pltpu.VMEM_SHARED((NPAD, 128), F32),
            pltpu.VMEM((CHUNK,), jnp.int32),
            pltpu.VMEM((CHUNK,), jnp.int32),
            pltpu.VMEM((CHUNK, 128), F32),
            pltpu.VMEM((CHUNK, 128), F32),
            pltpu.VMEM((TAIL,), jnp.int32),
            pltpu.VMEM((TAIL,), jnp.int32),
            pltpu.VMEM((TAIL, 128), F32),
            pltpu.VMEM((TAIL, 128), F32),
            pltpu.SemaphoreType.DMA,
        ],
    )(h, coef, central, neigh)


def _post_body(p0_ref, p1_ref, p2_ref, p3_ref, ne_ref, z_ref, w0e_ref,
               w1o_ref, wself_ref, s_ref, gx_ref, gy_ref, gz_ref):
    ta = (p0_ref[0, :, :] + p1_ref[0, :, :]
          + p2_ref[0, :, :] + p3_ref[0, :, :])
    tb = (p0_ref[1, :, :] + p1_ref[1, :, :]
          + p2_ref[1, :, :] + p3_ref[1, :, :])
    t0e = ta[:, :64]
    t1x = ta[:, 64:]
    t1y = tb[:, :64]
    t1z = tb[:, 64:]
    post0 = jnp.dot(t0e, w0e_ref[...], preferred_element_type=F32)
    si_all = jnp.dot(ne_ref[...], wself_ref[...], preferred_element_type=F32)
    z = z_ref[...]
    si = jnp.where(z == 0, si_all[:, 0:128], 0.0)
    si = si + jnp.where(z == 1, si_all[:, 128:256], 0.0)
    si = si + jnp.where(z == 2, si_all[:, 256:384], 0.0)
    si = si + jnp.where(z == 3, si_all[:, 384:512], 0.0)
    new0 = post0 + si
    sc = jax.nn.silu(new0[:, :64])
    g = jax.nn.silu(new0[:, 64:])
    w1o = w1o_ref[...]
    s_ref[...] = sc
    gx_ref[...] = g * jnp.dot(t1x, w1o, preferred_element_type=F32)
    gy_ref[...] = g * jnp.dot(t1y, w1o, preferred_element_type=F32)
    gz_ref[...] = g * jnp.dot(t1z, w1o, preferred_element_type=F32)


def kernel(node_embeddings, neighbour_distances, edge_embedding, edge_index,
           atomic_numbers, W_pre, bessel_freqs, W1, b1, W2, b2, W3, b3,
           W_post_0e, W_post_1o, W_self):
    central = edge_index[0]
    neigh = edge_index[1]

    h = pl.pallas_call(
        _pre_body,
        grid=(N // BN,),
        in_specs=[
            pl.BlockSpec((BN, 64), lambda i: (i, 0)),
            pl.BlockSpec((64, 64), lambda i: (0, 0)),
        ],
        out_specs=pl.BlockSpec((BN, 128), lambda i: (i, 0)),
        out_shape=jax.ShapeDtypeStruct((N, 128), F32),
    )(node_embeddings, W_pre * 0.125)

    w1p = jnp.zeros((128, 128), F32).at[:8, :10].set(W1)
    b1p = jnp.zeros((1, 128), F32).at[0, :10].set(b1)
    w2p = jnp.zeros((128, 128), F32).at[:10, :10].set(W2)
    b2p = jnp.zeros((1, 128), F32).at[0, :10].set(b2)
    w3p = jnp.zeros((128, 128), F32).at[:10, :].set(W3)
    b3p = b3.reshape(1, 128)
    freqs_p = jnp.zeros((1, 128), F32).at[0, :8].set(bessel_freqs / CUTOFF)
    nblk = E // BE
    r3 = neighbour_distances.reshape(nblk, 1, BE)
    e3 = edge_embedding.T.reshape(4, nblk, BE).transpose(1, 0, 2)
    packed = jnp.concatenate(
        [r3, e3, jnp.zeros((nblk, 3, BE), F32)], axis=1).reshape(8 * nblk, BE)
    nb = EP // BE

    def coef_part(i):
        return pl.pallas_call(
            _edge_body,
            grid=(nb,),
            in_specs=[
                pl.BlockSpec((8, BE), lambda j, i=i: (i * nb + j, 0)),
                pl.BlockSpec((1, 128), lambda j: (0, 0)),
                pl.BlockSpec((128, 128), lambda j: (0, 0)),
                pl.BlockSpec((1, 128), lambda j: (0, 0)),
                pl.BlockSpec((128, 128), lambda j: (0, 0)),
                pl.BlockSpec((1, 128), lambda j: (0, 0)),
                pl.BlockSpec((128, 128), lambda j: (0, 0)),
                pl.BlockSpec((1, 128), lambda j: (0, 0)),
            ],
            out_specs=pl.BlockSpec((NCORE, BE, 128), lambda j: (0, j, 0)),
            out_shape=jax.ShapeDtypeStruct((NCORE, EP, 128), F32),
        )(packed, freqs_p, w1p, b1p, w2p, b2p, w3p, b3p)

    parts = []
    for i in range(NPART):
        coef_i = coef_part(i)
        parts.append(_sc_call(h, coef_i, central, neigh, i * EP))

    w0e_s = W_post_0e * 0.125
    w1o_s = W_post_1o * 0.125
    wself_s = W_self.reshape(64, 512) * (1.0 / 16.0)
    z2 = atomic_numbers.reshape(N, 1)

    part_spec = pl.BlockSpec((NCORE, BN, 128), lambda i: (0, i, 0))
    outs = pl.pallas_call(
        _post_body,
        grid=(N // BN,),
        in_specs=[
            part_spec,
            part_spec,
            part_spec,
            part_spec,
            pl.BlockSpec((BN, 64), lambda i: (i, 0)),
            pl.BlockSpec((BN, 1), lambda i: (i, 0)),
            pl.BlockSpec((64, 128), lambda i: (0, 0)),
            pl.BlockSpec((64, 64), lambda i: (0, 0)),
            pl.BlockSpec((64, 512), lambda i: (0, 0)),
        ],
        out_specs=[
            pl.BlockSpec((BN, 64), lambda i: (i, 0)),
            pl.BlockSpec((BN, 64), lambda i: (i, 0)),
            pl.BlockSpec((BN, 64), lambda i: (i, 0)),
            pl.BlockSpec((BN, 64), lambda i: (i, 0)),
        ],
        out_shape=[jax.ShapeDtypeStruct((N, 64), F32)] * 4,
    )(parts[0], parts[1], parts[2], parts[3],
      node_embeddings, z2, w0e_s, w1o_s, wself_s)
    sc, gx, gy, gz = outs

    gated = jnp.stack([gx, gy, gz], axis=-1).reshape(N, 192)
    return jnp.concatenate([sc, gated], axis=1)


# speedup vs baseline: 2.7972x; 1.0346x over previous
import functools
import math

import jax
import jax.numpy as jnp
from jax import lax
from jax.experimental import pallas as pl
from jax.experimental.pallas import tpu as pltpu
from jax.experimental.pallas import tpu_sc as plsc

N = 10000
E = 320000
CUTOFF = 5.0
F32 = jnp.float32

NTILE = 16
NCORE = 2
NPART = 10
EP = E // NPART
EPT = EP // NTILE
CHUNK = 128
NFULL = EPT // CHUNK
TAIL = EPT - NFULL * CHUNK
NPAD = 10240
ROWS_PT = NPAD // NTILE
ROW_STEP = 128

BN = 2000
BE = 2000


def _pre_body(ne_ref, w_ref, h_ref):
    hh = jnp.dot(ne_ref[...], w_ref[...], preferred_element_type=F32)
    h_ref[:, 0:64] = hh
    h_ref[:, 64:128] = hh


def _edge_body(pk_ref, fp_ref, w1_ref, b1_ref, w2_ref, b2_ref,
               w3_ref, b3_ref, c_ref):
    t8 = jnp.transpose(pk_ref[...])
    r = t8[:, 0:1]
    s = jnp.sin(r * fp_ref[...])
    bes = (math.sqrt(2.0 / CUTOFF) * s) / r
    d = r * (1.0 / CUTOFF)
    d2 = d * d
    d6 = d2 * d2 * d2
    env = 1.0 - 28.0 * d6 + 48.0 * d6 * d - 21.0 * d6 * d2
    env = jnp.where(d < 1.0, env, 0.0)
    x = bes * env
    x = jax.nn.silu(jnp.dot(x, w1_ref[...], preferred_element_type=F32)
                    + b1_ref[...])
    x = jax.nn.silu(jnp.dot(x, w2_ref[...], preferred_element_type=F32)
                    + b2_ref[...])
    w = jnp.dot(x, w3_ref[...], preferred_element_type=F32) + b3_ref[...]
    w0 = w[:, :64]
    w1v = w[:, 64:]
    ee = t8[:, 1:5]
    c_ref[0, :, 0:64] = w0 * ee[:, 0:1]
    c_ref[0, :, 64:128] = w1v * ee[:, 1:2]
    c_ref[1, :, 0:64] = w1v * ee[:, 2:3]
    c_ref[1, :, 64:128] = w1v * ee[:, 3:4]


def _sc_body(pbase, h_hbm, c_hbm, cen_hbm, nei_hbm, t_hbm,
             acc, cidx, nidx, crows, msg,
             cidx_t, nidx_t, sem):
    c = lax.axis_index("c")
    s = lax.axis_index("s")

    def zrow(i, carry):
        for q in range(8):
            msg[i, pl.ds(q * 16, 16)] = jnp.zeros((16,), F32)
        return carry
    lax.fori_loop(0, CHUNK, zrow, 0)
    rb = s * ROWS_PT
    for j in range(ROWS_PT // ROW_STEP):
        pltpu.sync_copy(msg.at[pl.ds(0, ROW_STEP)],
                        acc.at[pl.ds(rb + j * ROW_STEP, ROW_STEP)])
    plsc.subcore_barrier()

    base0 = s * EPT

    def do_chunk(base, n, ci, ni):
        cr = crows.at[pl.ds(0, n)] if n != CHUNK else crows
        mg = msg.at[pl.ds(0, n)] if n != CHUNK else msg
        pltpu.sync_copy(cen_hbm.at[pl.ds(pbase + base, n)], ci)
        pltpu.sync_copy(nei_hbm.at[pl.ds(pbase + base, n)], ni)
        pltpu.async_copy(h_hbm.at[ni], mg, sem).wait()
        pltpu.sync_copy(c_hbm.at[c, pl.ds(base, n)], cr)

        def body(i2, carry):
            for d in range(2):
                e = i2 * 2 + d
                for q in range(4):
                    hq = mg[e, pl.ds(16 * q, 16)]
                    mg[e, pl.ds(16 * q, 16)] = cr[e, pl.ds(16 * q, 16)] * hq
                    mg[e, pl.ds(16 * q + 64, 16)] = (
                        cr[e, pl.ds(16 * q + 64, 16)] * hq)
            return carry
        lax.fori_loop(0, n // 2, body, 0)
        pltpu.sync_copy(mg, acc.at[ci], add=True)

    def chunk_loop(k, carry):
        do_chunk(base0 + k * CHUNK, CHUNK, cidx, nidx)
        return carry
    lax.fori_loop(0, NFULL, chunk_loop, 0)
    if TAIL:
        do_chunk(base0 + NFULL * CHUNK, TAIL, cidx_t, nidx_t)

    plsc.subcore_barrier()
    for j in range(ROWS_PT // ROW_STEP):
        pltpu.sync_copy(acc.at[pl.ds(rb + j * ROW_STEP, ROW_STEP)],
                        msg.at[pl.ds(0, ROW_STEP)])
        pltpu.sync_copy(msg.at[pl.ds(0, ROW_STEP)],
                        t_hbm.at[c, pl.ds(rb + j * ROW_STEP, ROW_STEP)])


def _sc_call(h, coef, central, neigh, pbase):
    mesh = plsc.VectorSubcoreMesh(core_axis_name="c", subcore_axis_name="s")
    return pl.kernel(
        functools.partial(_sc_body, pbase),
        out_type=jax.ShapeDtypeStruct((NCORE, NPAD, 128), F32),
        mesh=mesh,
        scratch_types=[
            pltpu.VMEM_SHARED((NPAD, 128), F32),
            pltpu.VMEM((CHUNK,), jnp.int32),
            pltpu.VMEM((CHUNK,), jnp.int32),
            pltpu.VMEM((CHUNK, 128), F32),
            pltpu.VMEM((CHUNK, 128), F32),
            pltpu.VMEM((TAIL,), jnp.int32),
            pltpu.VMEM((TAIL,), jnp.int32),
            pltpu.SemaphoreType.DMA,
        ],
    )(h, coef, central, neigh)


def _post_body(*refs):
    part_refs = refs[:NPART]
    (ne_ref, z_ref, w0e_ref, w1o_ref, wself_ref,
     s_ref, gx_ref, gy_ref, gz_ref) = refs[NPART:]
    ta = part_refs[0][0, :, :]
    tb = part_refs[0][1, :, :]
    for p in part_refs[1:]:
        ta = ta + p[0, :, :]
        tb = tb + p[1, :, :]
    t0e = ta[:, :64]
    t1x = ta[:, 64:]
    t1y = tb[:, :64]
    t1z = tb[:, 64:]
    post0 = jnp.dot(t0e, w0e_ref[...], preferred_element_type=F32)
    si_all = jnp.dot(ne_ref[...], wself_ref[...], preferred_element_type=F32)
    z = z_ref[...]
    si = jnp.where(z == 0, si_all[:, 0:128], 0.0)
    si = si + jnp.where(z == 1, si_all[:, 128:256], 0.0)
    si = si + jnp.where(z == 2, si_all[:, 256:384], 0.0)
    si = si + jnp.where(z == 3, si_all[:, 384:512], 0.0)
    new0 = post0 + si
    sc = jax.nn.silu(new0[:, :64])
    g = jax.nn.silu(new0[:, 64:])
    w1o = w1o_ref[...]
    s_ref[...] = sc
    gx_ref[...] = g * jnp.dot(t1x, w1o, preferred_element_type=F32)
    gy_ref[...] = g * jnp.dot(t1y, w1o, preferred_element_type=F32)
    gz_ref[...] = g * jnp.dot(t1z, w1o, preferred_element_type=F32)


def kernel(node_embeddings, neighbour_distances, edge_embedding, edge_index,
           atomic_numbers, W_pre, bessel_freqs, W1, b1, W2, b2, W3, b3,
           W_post_0e, W_post_1o, W_self):
    central = edge_index[0]
    neigh = edge_index[1]

    h = pl.pallas_call(
        _pre_body,
        grid=(N // BN,),
        in_specs=[
            pl.BlockSpec((BN, 64), lambda i: (i, 0)),
            pl.BlockSpec((64, 64), lambda i: (0, 0)),
        ],
        out_specs=pl.BlockSpec((BN, 128), lambda i: (i, 0)),
        out_shape=jax.ShapeDtypeStruct((N, 128), F32),
    )(node_embeddings, W_pre * 0.125)

    w1p = jnp.zeros((128, 128), F32).at[:8, :10].set(W1)
    b1p = jnp.zeros((1, 128), F32).at[0, :10].set(b1)
    w2p = jnp.zeros((128, 128), F32).at[:10, :10].set(W2)
    b2p = jnp.zeros((1, 128), F32).at[0, :10].set(b2)
    w3p = jnp.zeros((128, 128), F32).at[:10, :].set(W3)
    b3p = b3.reshape(1, 128)
    freqs_p = jnp.zeros((1, 128), F32).at[0, :8].set(bessel_freqs / CUTOFF)
    nblk = E // BE
    r3 = neighbour_distances.reshape(nblk, 1, BE)
    e3 = edge_embedding.T.reshape(4, nblk, BE).transpose(1, 0, 2)
    packed = jnp.concatenate(
        [r3, e3, jnp.zeros((nblk, 3, BE), F32)], axis=1).reshape(8 * nblk, BE)
    nb = EP // BE

    def coef_part(i):
        return pl.pallas_call(
            _edge_body,
            grid=(nb,),
            in_specs=[
                pl.BlockSpec((8, BE), lambda j, i=i: (i * nb + j, 0)),
                pl.BlockSpec((1, 128), lambda j: (0, 0)),
                pl.BlockSpec((128, 128), lambda j: (0, 0)),
                pl.BlockSpec((1, 128), lambda j: (0, 0)),
                pl.BlockSpec((128, 128), lambda j: (0, 0)),
                pl.BlockSpec((1, 128), lambda j: (0, 0)),
                pl.BlockSpec((128, 128), lambda j: (0, 0)),
                pl.BlockSpec((1, 128), lambda j: (0, 0)),
            ],
            out_specs=pl.BlockSpec((NCORE, BE, 128), lambda j: (0, j, 0)),
            out_shape=jax.ShapeDtypeStruct((NCORE, EP, 128), F32),
        )(packed, freqs_p, w1p, b1p, w2p, b2p, w3p, b3p)

    parts = []
    for i in range(NPART):
        coef_i = coef_part(i)
        parts.append(_sc_call(h, coef_i, central, neigh, i * EP))

    w0e_s = W_post_0e * 0.125
    w1o_s = W_post_1o * 0.125
    wself_s = W_self.reshape(64, 512) * (1.0 / 16.0)
    z2 = atomic_numbers.reshape(N, 1)

    part_spec = pl.BlockSpec((NCORE, BN, 128), lambda i: (0, i, 0))
    outs = pl.pallas_call(
        _post_body,
        grid=(N // BN,),
        in_specs=[part_spec] * NPART + [
            pl.BlockSpec((BN, 64), lambda i: (i, 0)),
            pl.BlockSpec((BN, 1), lambda i: (i, 0)),
            pl.BlockSpec((64, 128), lambda i: (0, 0)),
            pl.BlockSpec((64, 64), lambda i: (0, 0)),
            pl.BlockSpec((64, 512), lambda i: (0, 0)),
        ],
        out_specs=[
            pl.BlockSpec((BN, 64), lambda i: (i, 0)),
            pl.BlockSpec((BN, 64), lambda i: (i, 0)),
            pl.BlockSpec((BN, 64), lambda i: (i, 0)),
            pl.BlockSpec((BN, 64), lambda i: (i, 0)),
        ],
        out_shape=[jax.ShapeDtypeStruct((N, 64), F32)] * 4,
    )(*parts, node_embeddings, z2, w0e_s, w1o_s, wself_s)
    sc, gx, gy, gz = outs

    gated = jnp.stack([gx, gy, gz], axis=-1).reshape(N, 192)
    return jnp.concatenate([sc, gated], axis=1)


# speedup vs baseline: 2.9382x; 1.0504x over previous
import functools
import math

import jax
import jax.numpy as jnp
from jax import lax
from jax.experimental import pallas as pl
from jax.experimental.pallas import tpu as pltpu
from jax.experimental.pallas import tpu_sc as plsc

N = 10000
E = 320000
CUTOFF = 5.0
F32 = jnp.float32

NTILE = 16
NCORE = 2
NPART = 10
EP = E // NPART
EPT = EP // NTILE
CHUNK = 128
NFULL = EPT // CHUNK
TAIL = EPT - NFULL * CHUNK
NPAD = 10240
ROWS_PT = NPAD // NTILE
ROW_STEP = 128

BN = 2000
BE = 2000


def _pre_body(ne_ref, w_ref, h_ref):
    hh = jnp.dot(ne_ref[...], w_ref[...], preferred_element_type=F32)
    h_ref[:, 0:64] = hh
    h_ref[:, 64:128] = hh


def _edge_body(pk_ref, fp_ref, w1_ref, b1_ref, w2_ref, b2_ref,
               w3_ref, b3_ref, c_ref):
    pk = pk_ref[...]
    r_row = pk[0:1, :]
    rb = jnp.broadcast_to(r_row, (8, BE))
    s = jnp.sin(rb * fp_ref[...])
    bes = (math.sqrt(2.0 / CUTOFF) * s) / rb
    d = r_row * (1.0 / CUTOFF)
    d2 = d * d
    d6 = d2 * d2 * d2
    env = 1.0 - 28.0 * d6 + 48.0 * d6 * d - 21.0 * d6 * d2
    env = jnp.where(d < 1.0, env, 0.0)
    x8 = bes * jnp.broadcast_to(env, (8, BE))
    x = jnp.transpose(x8)
    w1s = w1_ref[...][0:8, :]
    x = jax.nn.silu(jnp.dot(x, w1s, preferred_element_type=F32)
                    + b1_ref[...])
    x = jax.nn.silu(jnp.dot(x, w2_ref[...], preferred_element_type=F32)
                    + b2_ref[...])
    w = jnp.dot(x, w3_ref[...], preferred_element_type=F32) + b3_ref[...]
    w0 = w[:, :64]
    w1v = w[:, 64:]
    ee = jnp.transpose(pk[1:5, :])
    c_ref[0, :, 0:64] = w0 * ee[:, 0:1]
    c_ref[0, :, 64:128] = w1v * ee[:, 1:2]
    c_ref[1, :, 0:64] = w1v * ee[:, 2:3]
    c_ref[1, :, 64:128] = w1v * ee[:, 3:4]


def _sc_body(pbase, h_hbm, c_hbm, cen_hbm, nei_hbm, t_hbm,
             acc, cidx, nidx, crows, msg,
             cidx_t, nidx_t, sem):
    c = lax.axis_index("c")
    s = lax.axis_index("s")

    def zrow(i, carry):
        for q in range(8):
            msg[i, pl.ds(q * 16, 16)] = jnp.zeros((16,), F32)
        return carry
    lax.fori_loop(0, CHUNK, zrow, 0)
    rb = s * ROWS_PT
    for j in range(ROWS_PT // ROW_STEP):
        pltpu.sync_copy(msg.at[pl.ds(0, ROW_STEP)],
                        acc.at[pl.ds(rb + j * ROW_STEP, ROW_STEP)])
    plsc.subcore_barrier()

    base0 = s * EPT

    def do_chunk(base, n, ci, ni):
        cr = crows.at[pl.ds(0, n)] if n != CHUNK else crows
        mg = msg.at[pl.ds(0, n)] if n != CHUNK else msg
        pltpu.sync_copy(cen_hbm.at[pl.ds(pbase + base, n)], ci)
        pltpu.sync_copy(nei_hbm.at[pl.ds(pbase + base, n)], ni)
        pltpu.async_copy(h_hbm.at[ni], mg, sem).wait()
        pltpu.sync_copy(c_hbm.at[c, pl.ds(base, n)], cr)

        def body(i2, carry):
            for d in range(2):
                e = i2 * 2 + d
                for q in range(4):
                    hq = mg[e, pl.ds(16 * q, 16)]
                    mg[e, pl.ds(16 * q, 16)] = cr[e, pl.ds(16 * q, 16)] * hq
                    mg[e, pl.ds(16 * q + 64, 16)] = (
                        cr[e, pl.ds(16 * q + 64, 16)] * hq)
            return carry
        lax.fori_loop(0, n // 2, body, 0)
        pltpu.sync_copy(mg, acc.at[ci], add=True)

    def chunk_loop(k, carry):
        do_chunk(base0 + k * CHUNK, CHUNK, cidx, nidx)
        return carry
    lax.fori_loop(0, NFULL, chunk_loop, 0)
    if TAIL:
        do_chunk(base0 + NFULL * CHUNK, TAIL, cidx_t, nidx_t)

    plsc.subcore_barrier()
    for j in range(ROWS_PT // ROW_STEP):
        pltpu.sync_copy(acc.at[pl.ds(rb + j * ROW_STEP, ROW_STEP)],
                        msg.at[pl.ds(0, ROW_STEP)])
        pltpu.sync_copy(msg.at[pl.ds(0, ROW_STEP)],
                        t_hbm.at[c, pl.ds(rb + j * ROW_STEP, ROW_STEP)])


def _sc_call(h, coef, central, neigh, pbase):
    mesh = plsc.VectorSubcoreMesh(core_axis_name="c", subcore_axis_name="s")
    return pl.kernel(
        functools.partial(_sc_body, pbase),
        out_type=jax.ShapeDtypeStruct((NCORE, NPAD, 128), F32),
        mesh=mesh,
        scratch_types=[
            pltpu.VMEM_SHARED((NPAD, 128), F32),
            pltpu.VMEM((CHUNK,), jnp.int32),
            pltpu.VMEM((CHUNK,), jnp.int32),
            pltpu.VMEM((CHUNK, 128), F32),
            pltpu.VMEM((CHUNK, 128), F32),
            pltpu.VMEM((TAIL,), jnp.int32),
            pltpu.VMEM((TAIL,), jnp.int32),
            pltpu.SemaphoreType.DMA,
        ],
    )(h, coef, central, neigh)


def _post_body(*refs):
    part_refs = refs[:NPART]
    (ne_ref, z_ref, w0e_ref, w1o_ref, wself_ref,
     s_ref, gx_ref, gy_ref, gz_ref) = refs[NPART:]
    ta = part_refs[0][0, :, :]
    tb = part_refs[0][1, :, :]
    for p in part_refs[1:]:
        ta = ta + p[0, :, :]
        tb = tb + p[1, :, :]
    t0e = ta[:, :64]
    t1x = ta[:, 64:]
    t1y = tb[:, :64]
    t1z = tb[:, 64:]
    post0 = jnp.dot(t0e, w0e_ref[...], preferred_element_type=F32)
    si_all = jnp.dot(ne_ref[...], wself_ref[...], preferred_element_type=F32)
    z = z_ref[...]
    si = jnp.where(z == 0, si_all[:, 0:128], 0.0)
    si = si + jnp.where(z == 1, si_all[:, 128:256], 0.0)
    si = si + jnp.where(z == 2, si_all[:, 256:384], 0.0)
    si = si + jnp.where(z == 3, si_all[:, 384:512], 0.0)
    new0 = post0 + si
    sc = jax.nn.silu(new0[:, :64])
    g = jax.nn.silu(new0[:, 64:])
    w1o = w1o_ref[...]
    s_ref[...] = sc
    gx_ref[...] = g * jnp.dot(t1x, w1o, preferred_element_type=F32)
    gy_ref[...] = g * jnp.dot(t1y, w1o, preferred_element_type=F32)
    gz_ref[...] = g * jnp.dot(t1z, w1o, preferred_element_type=F32)


def kernel(node_embeddings, neighbour_distances, edge_embedding, edge_index,
           atomic_numbers, W_pre, bessel_freqs, W1, b1, W2, b2, W3, b3,
           W_post_0e, W_post_1o, W_self):
    central = edge_index[0]
    neigh = edge_index[1]

    h = pl.pallas_call(
        _pre_body,
        grid=(N // BN,),
        in_specs=[
            pl.BlockSpec((BN, 64), lambda i: (i, 0)),
            pl.BlockSpec((64, 64), lambda i: (0, 0)),
        ],
        out_specs=pl.BlockSpec((BN, 128), lambda i: (i, 0)),
        out_shape=jax.ShapeDtypeStruct((N, 128), F32),
    )(node_embeddings, W_pre * 0.125)

    w1p = jnp.zeros((128, 128), F32).at[:8, :10].set(W1)
    b1p = jnp.zeros((1, 128), F32).at[0, :10].set(b1)
    w2p = jnp.zeros((128, 128), F32).at[:10, :10].set(W2)
    b2p = jnp.zeros((1, 128), F32).at[0, :10].set(b2)
    w3p = jnp.zeros((128, 128), F32).at[:10, :].set(W3)
    b3p = b3.reshape(1, 128)
    freqs_p = (bessel_freqs / CUTOFF).reshape(8, 1)
    nblk = E // BE
    r3 = neighbour_distances.reshape(nblk, 1, BE)
    e3 = edge_embedding.T.reshape(4, nblk, BE).transpose(1, 0, 2)
    packed = jnp.concatenate(
        [r3, e3, jnp.zeros((nblk, 3, BE), F32)], axis=1).reshape(8 * nblk, BE)
    nb = EP // BE

    def coef_part(i):
        return pl.pallas_call(
            _edge_body,
            grid=(nb,),
            in_specs=[
                pl.BlockSpec((8, BE), lambda j, i=i: (i * nb + j, 0)),
                pl.BlockSpec((8, 1), lambda j: (0, 0)),
                pl.BlockSpec((128, 128), lambda j: (0, 0)),
                pl.BlockSpec((1, 128), lambda j: (0, 0)),
                pl.BlockSpec((128, 128), lambda j: (0, 0)),
                pl.BlockSpec((1, 128), lambda j: (0, 0)),
                pl.BlockSpec((128, 128), lambda j: (0, 0)),
                pl.BlockSpec((1, 128), lambda j: (0, 0)),
            ],
            out_specs=pl.BlockSpec((NCORE, BE, 128), lambda j: (0, j, 0)),
            out_shape=jax.ShapeDtypeStruct((NCORE, EP, 128), F32),
        )(packed, freqs_p, w1p, b1p, w2p, b2p, w3p, b3p)

    parts = []
    for i in range(NPART):
        coef_i = coef_part(i)
        parts.append(_sc_call(h, coef_i, central, neigh, i * EP))

    w0e_s = W_post_0e * 0.125
    w1o_s = W_post_1o * 0.125
    wself_s = W_self.reshape(64, 512) * (1.0 / 16.0)
    z2 = atomic_numbers.reshape(N, 1)

    part_spec = pl.BlockSpec((NCORE, BN, 128), lambda i: (0, i, 0))
    outs = pl.pallas_call(
        _post_body,
        grid=(N // BN,),
        in_specs=[part_spec] * NPART + [
            pl.BlockSpec((BN, 64), lambda i: (i, 0)),
            pl.BlockSpec((BN, 1), lambda i: (i, 0)),
            pl.BlockSpec((64, 128), lambda i: (0, 0)),
            pl.BlockSpec((64, 64), lambda i: (0, 0)),
            pl.BlockSpec((64, 512), lambda i: (0, 0)),
        ],
        out_specs=[
            pl.BlockSpec((BN, 64), lambda i: (i, 0)),
            pl.BlockSpec((BN, 64), lambda i: (i, 0)),
            pl.BlockSpec((BN, 64), lambda i: (i, 0)),
            pl.BlockSpec((BN, 64), lambda i: (i, 0)),
        ],
        out_shape=[jax.ShapeDtypeStruct((N, 64), F32)] * 4,
    )(*parts, node_embeddings, z2, w0e_s, w1o_s, wself_s)
    sc, gx, gy, gz = outs

    gated = jnp.stack([gx, gy, gz], axis=-1).reshape(N, 192)
    return jnp.concatenate([sc, gated], axis=1)
